# baseline (speedup 1.0000x reference)
import functools

import jax
import jax.numpy as jnp
from jax import lax
from jax.experimental import pallas as pl
from jax.experimental.pallas import tpu as pltpu
from jax.experimental.pallas import tpu_sc as plsc

_N_ROWS = 1000000
_DIM = 64
_NUM_WORKERS = 32
_ROWS_PER_WORKER = 31248
_CHUNK_ROWS = 248
_CHUNKS_PER_TABLE = _ROWS_PER_WORKER // _CHUNK_ROWS
_TAIL_BASE = _NUM_WORKERS * _ROWS_PER_WORKER
_TAIL_ROWS = _N_ROWS - _TAIL_BASE
_N_BUF = 4
_LAG = 2

_mesh = plsc.VectorSubcoreMesh(core_axis_name="c", subcore_axis_name="s")


@functools.partial(
    pl.kernel,
    out_type=(
        jax.ShapeDtypeStruct((_N_ROWS, _DIM), jnp.float32),
        jax.ShapeDtypeStruct((_N_ROWS, _DIM), jnp.float32),
    ),
    mesh=_mesh,
    scratch_types=(
        [pltpu.VMEM_SHARED((16, _N_BUF, _CHUNK_ROWS, _DIM), jnp.float32)]
        + [pltpu.SemaphoreType.DMA] * (2 * _N_BUF)
    ),
)
def _sc_copy(u_hbm, i_hbm, out_u, out_i, shared, *sems):
    in_sems = sems[:_N_BUF]
    out_sems = sems[_N_BUF:]
    sid = lax.axis_index("s")
    bufs = tuple(shared.at[sid, b] for b in range(_N_BUF))
    wid = lax.axis_index("c") * 16 + sid
    base = wid * _ROWS_PER_WORKER

    tasks = []
    for k in range(_CHUNKS_PER_TABLE):
        tasks.append((u_hbm, out_u, k))
        tasks.append((i_hbm, out_i, k))

    def in_copy(t):
        src, _, k = tasks[t]
        slot = t % _N_BUF
        return pltpu.make_async_copy(
            src.at[pl.ds(base + k * _CHUNK_ROWS, _CHUNK_ROWS), :],
            bufs[slot],
            in_sems[slot],
        )

    def out_copy(t):
        _, dst, k = tasks[t]
        slot = t % _N_BUF
        return pltpu.make_async_copy(
            bufs[slot],
            dst.at[pl.ds(base + k * _CHUNK_ROWS, _CHUNK_ROWS), :],
            out_sems[slot],
        )

    T = len(tasks)
    for u in range(T + _LAG):
        if u < T:
            if u >= _N_BUF:
                out_copy(u - _N_BUF).wait()
            in_copy(u).start()
        t_out = u - _LAG
        if 0 <= t_out < T:
            in_copy(t_out).wait()
            out_copy(t_out).start()
    for t in range(max(T - _N_BUF, 0), T):
        out_copy(t).wait()

    @pl.when(wid == 0)
    def _():
        for src, dst, slot in ((u_hbm, out_u, 0), (i_hbm, out_i, 1)):
            pltpu.make_async_copy(
                src.at[pl.ds(_TAIL_BASE, _TAIL_ROWS), :],
                bufs[slot].at[pl.ds(0, _TAIL_ROWS), :],
                in_sems[slot],
            ).start()
        for src, dst, slot in ((u_hbm, out_u, 0), (i_hbm, out_i, 1)):
            pltpu.make_async_copy(
                src.at[pl.ds(_TAIL_BASE, _TAIL_ROWS), :],
                bufs[slot].at[pl.ds(0, _TAIL_ROWS), :],
                in_sems[slot],
            ).wait()
            pltpu.make_async_copy(
                bufs[slot].at[pl.ds(0, _TAIL_ROWS), :],
                dst.at[pl.ds(_TAIL_BASE, _TAIL_ROWS), :],
                out_sems[slot],
            ).start()
        for src, dst, slot in ((u_hbm, out_u, 0), (i_hbm, out_i, 1)):
            pltpu.make_async_copy(
                bufs[slot].at[pl.ds(0, _TAIL_ROWS), :],
                dst.at[pl.ds(_TAIL_BASE, _TAIL_ROWS), :],
                out_sems[slot],
            ).wait()


_TC_BLOCK = 10000
_TC_GRID = _N_ROWS // _TC_BLOCK


def _tc_body(u_ref, i_ref, ou_ref, oi_ref):
    ou_ref[...] = u_ref[...]
    oi_ref[...] = i_ref[...]


_tc_copy = pl.pallas_call(
    _tc_body,
    grid=(_TC_GRID,),
    in_specs=[pl.BlockSpec((_TC_BLOCK, _DIM), lambda i: (i, 0))] * 2,
    out_specs=[pl.BlockSpec((_TC_BLOCK, _DIM), lambda i: (i, 0))] * 2,
    out_shape=(
        jax.ShapeDtypeStruct((_N_ROWS, _DIM), jnp.float32),
        jax.ShapeDtypeStruct((_N_ROWS, _DIM), jnp.float32),
    ),
    compiler_params=pltpu.CompilerParams(
        dimension_semantics=("parallel",),
    ),
)


def kernel(user_history, user_table, item_table):
    del user_history
    user_emb, item_emb = _tc_copy(user_table, item_table)
    return (user_emb, item_emb)


# speedup vs baseline: 1.0307x; 1.0307x over previous
import functools

import jax
import jax.numpy as jnp
from jax import lax
from jax.experimental import pallas as pl
from jax.experimental.pallas import tpu as pltpu
from jax.experimental.pallas import tpu_sc as plsc

_N_ROWS = 1000000
_DIM = 64
_NUM_WORKERS = 32
_ROWS_PER_WORKER = 31248
_CHUNK_ROWS = 248
_CHUNKS_PER_TABLE = _ROWS_PER_WORKER // _CHUNK_ROWS
_TAIL_BASE = _NUM_WORKERS * _ROWS_PER_WORKER
_TAIL_ROWS = _N_ROWS - _TAIL_BASE
_N_BUF = 4
_LAG = 2

_mesh = plsc.VectorSubcoreMesh(core_axis_name="c", subcore_axis_name="s")


@functools.partial(
    pl.kernel,
    out_type=jax.ShapeDtypeStruct((_N_ROWS, _DIM), jnp.float32),
    mesh=_mesh,
    scratch_types=(
        [pltpu.VMEM_SHARED((16, _N_BUF, _CHUNK_ROWS, _DIM), jnp.float32)]
        + [pltpu.SemaphoreType.DMA] * (2 * _N_BUF)
    ),
)
def _sc_copy(u_hbm, out_u, shared, *sems):
    in_sems = sems[:_N_BUF]
    out_sems = sems[_N_BUF:]
    sid = lax.axis_index("s")
    bufs = tuple(shared.at[sid, b] for b in range(_N_BUF))
    wid = lax.axis_index("c") * 16 + sid
    base = wid * _ROWS_PER_WORKER

    tasks = [(u_hbm, out_u, k) for k in range(_CHUNKS_PER_TABLE)]

    def in_copy(t):
        src, _, k = tasks[t]
        slot = t % _N_BUF
        return pltpu.make_async_copy(
            src.at[pl.ds(base + k * _CHUNK_ROWS, _CHUNK_ROWS), :],
            bufs[slot],
            in_sems[slot],
        )

    def out_copy(t):
        _, dst, k = tasks[t]
        slot = t % _N_BUF
        return pltpu.make_async_copy(
            bufs[slot],
            dst.at[pl.ds(base + k * _CHUNK_ROWS, _CHUNK_ROWS), :],
            out_sems[slot],
        )

    T = len(tasks)
    for u in range(T + _LAG):
        if u < T:
            if u >= _N_BUF:
                out_copy(u - _N_BUF).wait()
            in_copy(u).start()
        t_out = u - _LAG
        if 0 <= t_out < T:
            in_copy(t_out).wait()
            out_copy(t_out).start()
    for t in range(max(T - _N_BUF, 0), T):
        out_copy(t).wait()

    @pl.when(wid == 0)
    def _():
        pltpu.make_async_copy(
            u_hbm.at[pl.ds(_TAIL_BASE, _TAIL_ROWS), :],
            bufs[0].at[pl.ds(0, _TAIL_ROWS), :],
            in_sems[0],
        ).start()
        pltpu.make_async_copy(
            u_hbm.at[pl.ds(_TAIL_BASE, _TAIL_ROWS), :],
            bufs[0].at[pl.ds(0, _TAIL_ROWS), :],
            in_sems[0],
        ).wait()
        pltpu.make_async_copy(
            bufs[0].at[pl.ds(0, _TAIL_ROWS), :],
            out_u.at[pl.ds(_TAIL_BASE, _TAIL_ROWS), :],
            out_sems[0],
        ).start()
        pltpu.make_async_copy(
            bufs[0].at[pl.ds(0, _TAIL_ROWS), :],
            out_u.at[pl.ds(_TAIL_BASE, _TAIL_ROWS), :],
            out_sems[0],
        ).wait()


_TC_BLOCK = 10000
_TC_GRID = _N_ROWS // _TC_BLOCK


def _tc_body(i_ref, oi_ref):
    oi_ref[...] = i_ref[...]


_tc_copy = pl.pallas_call(
    _tc_body,
    grid=(_TC_GRID,),
    in_specs=[pl.BlockSpec((_TC_BLOCK, _DIM), lambda i: (i, 0))],
    out_specs=pl.BlockSpec((_TC_BLOCK, _DIM), lambda i: (i, 0)),
    out_shape=jax.ShapeDtypeStruct((_N_ROWS, _DIM), jnp.float32),
    compiler_params=pltpu.CompilerParams(
        dimension_semantics=("parallel",),
    ),
)


def kernel(user_history, user_table, item_table):
    del user_history
    user_emb = _sc_copy(user_table)
    item_emb = _tc_copy(item_table)
    return (user_emb, item_emb)
